# baseline (speedup 1.0000x reference)
import functools

import jax
import jax.numpy as jnp
from jax import lax
from jax.experimental import pallas as pl
from jax.experimental.pallas import tpu as pltpu
from jax.experimental.pallas import tpu_sc as plsc

NUM_CLASSES = 100000
EMBED_DIM = 64
BATCH = 16384

_NC = 2
_NS = 16
_NW = _NC * _NS
_BPW = BATCH // _NW
_ROW_STRIDE = NUM_CLASSES + 1
_FIRE = 16


def _gather_body(flat_hbm, labels_hbm, outT_hbm, idx_v, flat_v, rows_v, sem):
    wid = lax.axis_index("s") * _NC + lax.axis_index("c")
    base = wid * _BPW
    pltpu.sync_copy(labels_hbm.at[pl.ds(base, _BPW)], idx_v)

    def build(i, _):
        d = i // (_BPW // 16)
        j = (i % (_BPW // 16)) * 16
        flat_v[d, pl.ds(j, 16)] = idx_v[pl.ds(j, 16)] + d * _ROW_STRIDE
        return 0

    lax.fori_loop(0, EMBED_DIM * (_BPW // 16), build, 0)

    for g in range(0, EMBED_DIM, _FIRE):
        copies = [
            pltpu.async_copy(flat_hbm.at[flat_v.at[d]], rows_v.at[d], sem)
            for d in range(g, g + _FIRE)
        ]
        for c in copies:
            c.wait()

    pltpu.sync_copy(rows_v, outT_hbm.at[:, pl.ds(base, _BPW)])


@jax.jit
def _embed(labels, table):
    flat = jnp.reshape(table.T, (-1,))
    mesh = plsc.VectorSubcoreMesh(core_axis_name="c", subcore_axis_name="s")
    call = pl.kernel(
        _gather_body,
        out_type=jax.ShapeDtypeStruct((EMBED_DIM, BATCH), jnp.float32),
        mesh=mesh,
        scratch_types=[
            pltpu.VMEM((_BPW,), jnp.int32),
            pltpu.VMEM((EMBED_DIM, _BPW), jnp.int32),
            pltpu.VMEM((EMBED_DIM, _BPW), jnp.float32),
            pltpu.SemaphoreType.DMA,
        ],
        compiler_params=pltpu.CompilerParams(use_tc_tiling_on_sc=False),
    )
    outT = call(flat, labels)
    return outT.T


def kernel(labels, table):
    labels = jnp.asarray(labels, dtype=jnp.int32)
    if labels.ndim == 0:
        labels = labels[None]
    return _embed(labels, table)


# speedup vs baseline: 2.4879x; 2.4879x over previous
import functools

import jax
import jax.numpy as jnp
from jax import lax
from jax.experimental import pallas as pl
from jax.experimental.pallas import tpu as pltpu
from jax.experimental.pallas import tpu_sc as plsc

NUM_CLASSES = 100000
EMBED_DIM = 64
BATCH = 16384

_NC = 2
_NS = 16
_BSC = BATCH // _NC
_BPW = _BSC // _NS
_C = 1280
_P = 5
_NCOLS = NUM_CLASSES + 1
_PAD_COLS = 100096
_CLAMP_C0 = _PAD_COLS - _C
_SW = 128
_TRASH = BATCH
_CHUNK = 128


def _body(tbl, labels_hbm, out_hbm, stage_hbm, labels_v, slab, mlist, mbuf, bidx, rbuf, obuf, gidx):
    cid = lax.axis_index("c")
    sid = lax.axis_index("s")
    iota = lax.iota(jnp.int32, 16)

    pltpu.sync_copy(labels_hbm.at[pl.ds(cid * _BSC, _BSC)], labels_v)

    d_idx = [iota + 16 * k for k in range(4)]

    def do_pass(p, _):
        c0 = (p * _NS + sid) * _C
        c0c = jnp.minimum(c0, _CLAMP_C0)

        @pl.when(c0 < _NCOLS)
        def _():
            pltpu.sync_copy(tbl.at[:, pl.ds(c0c, _C)], slab)
            c1 = c0 + _C

            def scan(j, ptr):
                lv = labels_v[pl.ds(j * 16, 16)]
                m = (lv >= c0) & (lv < c1)
                packed = ((iota + j * 16) << 17) | lv
                mi = m.astype(jnp.int32)
                pos = plsc.cumsum(mi) - mi
                slot = jnp.where(m, ptr + pos, _BSC + 8)
                plsc.store_scatter(mlist, [slot], packed)
                return ptr + jnp.sum(mi)

            cnt = lax.fori_loop(0, _BSC // 16, scan, jnp.int32(0))

            def serve(g):
                pv = mlist[pl.ds(g, 16)]
                b_vec = cid * _BSC + (pv >> 17)
                lane_ok = (iota + g) < cnt
                bidx[...] = jnp.where(lane_ok, b_vec, _TRASH) * 8
                for m in range(16):
                    pm = plsc.load_gather(mlist, [jnp.full((16,), m, jnp.int32) + g])
                    c_loc = jnp.clip((pm & 0x1FFFF) - c0c, 0, _C - 1)
                    for k in range(4):
                        v = plsc.load_gather(slab, [d_idx[k], c_loc])
                        mbuf[m, pl.ds(16 * k, 16)] = v
                pltpu.sync_copy(mbuf, stage_hbm.at[bidx])
                return g + 16

            lax.while_loop(lambda g: g < cnt, serve, jnp.int32(0))

        return 0

    lax.fori_loop(0, _P, do_pass, 0)
    plsc.subcore_barrier()

    base_out = cid * _BSC + sid * _BPW
    for k in range(_BPW // _CHUNK):
        for t in range(_CHUNK // 16):
            gidx[pl.ds(t * 16, 16)] = (base_out + k * _CHUNK + t * 16 + iota) * 8
        pltpu.sync_copy(stage_hbm.at[gidx], rbuf)

        def transpose(i, _):
            d = i // (_CHUNK // 16)
            j = i % (_CHUNK // 16)
            dv = jnp.full((16,), d, jnp.int32)
            v = plsc.load_gather(rbuf, [j * 16 + iota, dv])
            plsc.store_scatter(obuf, [dv, j * 16 + iota], v)
            return 0

        lax.fori_loop(0, EMBED_DIM * (_CHUNK // 16), transpose, 0)
        pltpu.sync_copy(obuf, out_hbm.at[:, pl.ds(base_out + k * _CHUNK, _CHUNK)])


@jax.jit
def _embed(labels, table):
    tbl = table.T
    mesh = plsc.VectorSubcoreMesh(core_axis_name="c", subcore_axis_name="s")
    call = pl.kernel(
        _body,
        out_type=(
            jax.ShapeDtypeStruct((EMBED_DIM, BATCH), jnp.float32),
            jax.ShapeDtypeStruct(((BATCH + 8) * 8, _SW), jnp.float32),
        ),
        mesh=mesh,
        scratch_types=[
            pltpu.VMEM((_BSC,), jnp.int32),
            pltpu.VMEM((EMBED_DIM, _C), jnp.float32),
            pltpu.VMEM((_BSC + 16,), jnp.int32),
            pltpu.VMEM((16, _SW), jnp.float32),
            pltpu.VMEM((16,), jnp.int32),
            pltpu.VMEM((_CHUNK, _SW), jnp.float32),
            pltpu.VMEM((EMBED_DIM, _CHUNK), jnp.float32),
            pltpu.VMEM((_CHUNK,), jnp.int32),
        ],
        compiler_params=pltpu.CompilerParams(
            use_tc_tiling_on_sc=True, needs_layout_passes=False
        ),
    )
    outT, _ = call(tbl, labels)
    return outT.T


def kernel(labels, table):
    labels = jnp.asarray(labels, dtype=jnp.int32)
    if labels.ndim == 0:
        labels = labels[None]
    return _embed(labels, table)


# speedup vs baseline: 3.6958x; 1.4855x over previous
import functools

import jax
import jax.numpy as jnp
from jax import lax
from jax.experimental import pallas as pl
from jax.experimental.pallas import tpu as pltpu
from jax.experimental.pallas import tpu_sc as plsc

NUM_CLASSES = 100000
EMBED_DIM = 64
BATCH = 16384

_NC = 2
_NS = 16
_NW = _NC * _NS
_B_PER_W = BATCH // _NW


def _gather_body(table_hbm, labels_hbm, out_hbm, idx_v, rows_v, sem):
    wid = lax.axis_index("s") * _NC + lax.axis_index("c")
    base = wid * _B_PER_W
    pltpu.sync_copy(labels_hbm.at[pl.ds(base, _B_PER_W)], idx_v)
    pltpu.async_copy(table_hbm.at[idx_v], rows_v, sem).wait()
    pltpu.sync_copy(rows_v, out_hbm.at[pl.ds(base, _B_PER_W)])


@jax.jit
def _embed(labels, table):
    mesh = plsc.VectorSubcoreMesh(core_axis_name="c", subcore_axis_name="s")
    call = pl.kernel(
        _gather_body,
        out_type=jax.ShapeDtypeStruct((BATCH, EMBED_DIM), jnp.float32),
        mesh=mesh,
        scratch_types=[
            pltpu.VMEM((_B_PER_W,), jnp.int32),
            pltpu.VMEM((_B_PER_W, EMBED_DIM), jnp.float32),
            pltpu.SemaphoreType.DMA,
        ],
        compiler_params=pltpu.CompilerParams(use_tc_tiling_on_sc=False),
    )
    return call(table, labels)


def kernel(labels, table):
    labels = jnp.asarray(labels, dtype=jnp.int32)
    if labels.ndim == 0:
        labels = labels[None]
    return _embed(labels, table)
